# Initial kernel scaffold; baseline (speedup 1.0000x reference)
import functools

import jax
import jax.numpy as jnp
from jax import lax
from jax.experimental import pallas as pl
from jax.experimental.pallas import tpu as pltpu
from jax.experimental.pallas import tpu_sc as plsc

N_NODES = 10000
NOUT = 64

NUM_CORES = 2
NUM_SUBCORES = 16
NW = NUM_CORES * NUM_SUBCORES
CHUNK = 128
ROWS_PER_TILE = N_NODES // NUM_SUBCORES
WB = 125


def _dense_z(x, W1, b1, W2):
    def body(x_ref, w1_ref, b1_ref, w2_ref, z_ref):
        m = jnp.dot(w1_ref[...], w2_ref[...], preferred_element_type=jnp.float32)
        v = jnp.dot(b1_ref[...], w2_ref[...], preferred_element_type=jnp.float32)
        z_ref[...] = jnp.dot(x_ref[...], m, preferred_element_type=jnp.float32) + v

    return pl.pallas_call(
        body,
        out_shape=jax.ShapeDtypeStruct((x.shape[0], W2.shape[1]), jnp.float32),
    )(x, W1, b1.reshape(1, -1), W2)


def _sc_segsum(z, src_r, dst_r, w_r, nchunk):
    mesh = plsc.VectorSubcoreMesh(core_axis_name="c", subcore_axis_name="s")

    @functools.partial(
        pl.kernel,
        out_type=jax.ShapeDtypeStruct((NUM_CORES, N_NODES, NOUT), jnp.float32),
        mesh=mesh,
        scratch_types=[
            pltpu.VMEM((nchunk, CHUNK), jnp.int32),
            pltpu.VMEM((nchunk, CHUNK), jnp.int32),
            pltpu.VMEM((nchunk, CHUNK), jnp.float32),
            pltpu.VMEM((CHUNK, NOUT), jnp.float32),
            pltpu.VMEM((WB, NOUT), jnp.float32),
            pltpu.VMEM_SHARED((N_NODES, NOUT), jnp.float32),
            pltpu.SemaphoreType.DMA,
        ],
    )
    def k(z_hbm, src_hbm, dst_hbm, w_hbm, out_hbm,
          src_v, dst_v, w_v, rows_v, zero_v, acc_sh, sem):
        cid = lax.axis_index("c")
        sid = lax.axis_index("s")
        tid = cid * NUM_SUBCORES + sid

        @pl.loop(0, WB)
        def _zero_rows(i):
            for g in range(NOUT // 16):
                zero_v[i, pl.ds(g * 16, 16)] = jnp.zeros((16,), jnp.float32)

        row0 = sid * ROWS_PER_TILE
        for r in range(ROWS_PER_TILE // WB):
            pltpu.sync_copy(zero_v, acc_sh.at[pl.ds(row0 + r * WB, WB)])
        plsc.subcore_barrier()

        pltpu.sync_copy(src_hbm.at[tid], src_v)
        pltpu.sync_copy(dst_hbm.at[tid], dst_v)
        pltpu.sync_copy(w_hbm.at[tid], w_v)

        @pl.loop(0, nchunk)
        def _chunks(j):
            pltpu.async_copy(z_hbm.at[src_v.at[j]], rows_v, sem).wait()

            @pl.loop(0, CHUNK)
            def _scale(e):
                ws = w_v[j, e]
                for g in range(NOUT // 16):
                    sl = pl.ds(g * 16, 16)
                    rows_v[e, sl] = rows_v[e, sl] * ws

            pltpu.sync_copy(rows_v, acc_sh.at[dst_v.at[j]], add=True)

        plsc.subcore_barrier()
        for r in range(ROWS_PER_TILE // WB):
            sl = pl.ds(row0 + r * WB, WB)
            pltpu.sync_copy(acc_sh.at[sl], out_hbm.at[cid, sl])

    return k(z, src_r, dst_r, w_r)


def _combine(acc, b2):
    def body(a_ref, b2_ref, o_ref):
        o_ref[...] = a_ref[0] + a_ref[1] + b2_ref[...]

    return pl.pallas_call(
        body,
        out_shape=jax.ShapeDtypeStruct((N_NODES, NOUT), jnp.float32),
    )(acc, b2.reshape(1, -1))


def kernel(x, edge_index, edge_weight, W1, b1, W2, b2):
    e = edge_weight.shape[0]
    per_tile = -(-e // (NW * CHUNK)) * CHUNK
    pad = per_tile * NW - e
    src = edge_index[1].astype(jnp.int32)
    dst = edge_index[0].astype(jnp.int32)
    w = edge_weight.astype(jnp.float32)
    if pad:
        src = jnp.concatenate([src, jnp.zeros((pad,), jnp.int32)])
        dst = jnp.concatenate([dst, jnp.zeros((pad,), jnp.int32)])
        w = jnp.concatenate([w, jnp.zeros((pad,), jnp.float32)])
    nchunk = per_tile // CHUNK
    src_r = src.reshape(NW, nchunk, CHUNK)
    dst_r = dst.reshape(NW, nchunk, CHUNK)
    w_r = w.reshape(NW, nchunk, CHUNK)

    z = _dense_z(x, W1, b1, W2)
    acc = _sc_segsum(z, src_r, dst_r, w_r, nchunk)
    return _combine(acc, b2)


# speedup vs baseline: 6.2531x; 6.2531x over previous
import functools

import jax
import jax.numpy as jnp
from jax import lax
from jax.experimental import pallas as pl
from jax.experimental.pallas import tpu as pltpu
from jax.experimental.pallas import tpu_sc as plsc

N_NODES = 10000
NOUT = 64

NUM_CORES = 2
NUM_SUBCORES = 16
NW = NUM_CORES * NUM_SUBCORES
CHUNK = 128
N_PAD = 10240
ROWS_PER_TILE = N_PAD // NUM_SUBCORES
WB = 128


def _dense_z(x, W1, b1, W2):
    def body(x_ref, w1_ref, b1_ref, w2_ref, z_ref):
        m = jnp.dot(w1_ref[...], w2_ref[...], preferred_element_type=jnp.float32)
        v = jnp.dot(b1_ref[...], w2_ref[...], preferred_element_type=jnp.float32)
        z_ref[...] = jnp.dot(x_ref[...], m, preferred_element_type=jnp.float32) + v

    return pl.pallas_call(
        body,
        out_shape=jax.ShapeDtypeStruct((x.shape[0], W2.shape[1]), jnp.float32),
    )(x, W1, b1.reshape(1, -1), W2)


def _sc_segsum(z, src_r, dst_r, w_r, nchunk):
    mesh = plsc.VectorSubcoreMesh(core_axis_name="c", subcore_axis_name="s")

    @functools.partial(
        pl.kernel,
        out_type=jax.ShapeDtypeStruct((NUM_CORES, N_PAD, NOUT), jnp.float32),
        mesh=mesh,
        scratch_types=[
            pltpu.VMEM((nchunk, CHUNK), jnp.int32),
            pltpu.VMEM((nchunk, CHUNK), jnp.int32),
            pltpu.VMEM((nchunk, CHUNK), jnp.float32),
            pltpu.VMEM((CHUNK, NOUT), jnp.float32),
            pltpu.VMEM((WB, NOUT), jnp.float32),
            pltpu.VMEM_SHARED((N_PAD, NOUT), jnp.float32),
            pltpu.SemaphoreType.DMA,
        ],
        compiler_params=pltpu.CompilerParams(use_tc_tiling_on_sc=False),
    )
    def k(z_hbm, src_hbm, dst_hbm, w_hbm, out_hbm,
          src_v, dst_v, w_v, rows_v, zero_v, acc_sh, sem):
        cid = lax.axis_index("c")
        sid = lax.axis_index("s")
        tid = cid * NUM_SUBCORES + sid

        @pl.loop(0, WB)
        def _zero_rows(i):
            for g in range(NOUT // 16):
                zero_v[i, pl.ds(g * 16, 16)] = jnp.zeros((16,), jnp.float32)

        row0 = sid * ROWS_PER_TILE
        for r in range(ROWS_PER_TILE // WB):
            pltpu.sync_copy(zero_v, acc_sh.at[pl.ds(row0 + r * WB, WB)])
        plsc.subcore_barrier()

        pltpu.sync_copy(src_hbm.at[tid], src_v)
        pltpu.sync_copy(dst_hbm.at[tid], dst_v)
        pltpu.sync_copy(w_hbm.at[tid], w_v)

        @pl.loop(0, nchunk)
        def _chunks(j):
            pltpu.async_copy(z_hbm.at[src_v.at[j]], rows_v, sem).wait()

            @pl.loop(0, CHUNK // 16)
            def _scale(eg):
                wv = w_v[j, pl.ds(eg * 16, 16)]
                for l in range(16):
                    ws = wv[l]
                    e = eg * 16 + l
                    for g in range(NOUT // 16):
                        sl = pl.ds(g * 16, 16)
                        rows_v[e, sl] = rows_v[e, sl] * ws

            pltpu.sync_copy(rows_v, acc_sh.at[dst_v.at[j]], add=True)

        plsc.subcore_barrier()
        for r in range(ROWS_PER_TILE // WB):
            sl = pl.ds(row0 + r * WB, WB)
            pltpu.sync_copy(acc_sh.at[sl], out_hbm.at[cid, sl])

    return k(z, src_r, dst_r, w_r)


def _combine(acc, b2):
    def body(a_ref, b2_ref, o_ref):
        o_ref[...] = a_ref[0, :N_NODES] + a_ref[1, :N_NODES] + b2_ref[...]

    return pl.pallas_call(
        body,
        out_shape=jax.ShapeDtypeStruct((N_NODES, NOUT), jnp.float32),
    )(acc, b2.reshape(1, -1))


def kernel(x, edge_index, edge_weight, W1, b1, W2, b2):
    e = edge_weight.shape[0]
    per_tile = -(-e // (NW * CHUNK)) * CHUNK
    pad = per_tile * NW - e
    src = edge_index[1].astype(jnp.int32)
    dst = edge_index[0].astype(jnp.int32)
    w = edge_weight.astype(jnp.float32)
    if pad:
        src = jnp.concatenate([src, jnp.zeros((pad,), jnp.int32)])
        dst = jnp.concatenate([dst, jnp.zeros((pad,), jnp.int32)])
        w = jnp.concatenate([w, jnp.zeros((pad,), jnp.float32)])
    nchunk = per_tile // CHUNK
    src_r = src.reshape(NW, nchunk, CHUNK)
    dst_r = dst.reshape(NW, nchunk, CHUNK)
    w_r = w.reshape(NW, nchunk, CHUNK)

    z = _dense_z(x, W1, b1, W2)
    acc = _sc_segsum(z, src_r, dst_r, w_r, nchunk)
    return _combine(acc, b2)


# speedup vs baseline: 7.8731x; 1.2591x over previous
import functools

import jax
import jax.numpy as jnp
from jax import lax
from jax.experimental import pallas as pl
from jax.experimental.pallas import tpu as pltpu
from jax.experimental.pallas import tpu_sc as plsc

N_NODES = 10000
NOUT = 64

NUM_CORES = 2
NUM_SUBCORES = 16
NW = NUM_CORES * NUM_SUBCORES
CHUNK = 128
N_PAD = 10240
ROWS_PER_TILE = N_PAD // NUM_SUBCORES
WB = 128


def _dense_z(x, W1, b1, W2):
    def body(x_ref, w1_ref, b1_ref, w2_ref, z_ref):
        m = jnp.dot(w1_ref[...], w2_ref[...], preferred_element_type=jnp.float32)
        v = jnp.dot(b1_ref[...], w2_ref[...], preferred_element_type=jnp.float32)
        z_ref[...] = jnp.dot(x_ref[...], m, preferred_element_type=jnp.float32) + v

    return pl.pallas_call(
        body,
        out_shape=jax.ShapeDtypeStruct((x.shape[0], W2.shape[1]), jnp.float32),
    )(x, W1, b1.reshape(1, -1), W2)


NBUF = 4


def _sc_segsum(z, src_r, dst_r, w_r, nchunk):
    mesh = plsc.VectorSubcoreMesh(core_axis_name="c", subcore_axis_name="s")
    assert nchunk % NBUF == 0

    @functools.partial(
        pl.kernel,
        out_type=jax.ShapeDtypeStruct((NUM_CORES, N_PAD, NOUT), jnp.float32),
        mesh=mesh,
        scratch_types=[
            pltpu.VMEM((nchunk, CHUNK), jnp.int32),
            pltpu.VMEM((nchunk, CHUNK), jnp.int32),
            pltpu.VMEM((nchunk, CHUNK), jnp.float32),
            pltpu.VMEM((NBUF, CHUNK, NOUT), jnp.float32),
            pltpu.VMEM((WB, NOUT), jnp.float32),
            pltpu.VMEM_SHARED((N_PAD, NOUT), jnp.float32),
            [pltpu.SemaphoreType.DMA] * NBUF,
        ],
        compiler_params=pltpu.CompilerParams(use_tc_tiling_on_sc=False),
    )
    def k(z_hbm, src_hbm, dst_hbm, w_hbm, out_hbm,
          src_v, dst_v, w_v, rows_v, zero_v, acc_sh, gsems):
        cid = lax.axis_index("c")
        sid = lax.axis_index("s")
        tid = cid * NUM_SUBCORES + sid

        @pl.loop(0, WB)
        def _zero_rows(i):
            for g in range(NOUT // 16):
                zero_v[i, pl.ds(g * 16, 16)] = jnp.zeros((16,), jnp.float32)

        row0 = sid * ROWS_PER_TILE
        for r in range(ROWS_PER_TILE // WB):
            pltpu.sync_copy(zero_v, acc_sh.at[pl.ds(row0 + r * WB, WB)])
        plsc.subcore_barrier()

        pltpu.sync_copy(src_hbm.at[tid], src_v)
        pltpu.sync_copy(dst_hbm.at[tid], dst_v)
        pltpu.sync_copy(w_hbm.at[tid], w_v)

        for b in range(NBUF - 1):
            pltpu.async_copy(z_hbm.at[src_v.at[b]], rows_v.at[b], gsems[b])

        @pl.loop(0, nchunk // NBUF)
        def _groups(g):
            for b in range(NBUF):
                j = g * NBUF + b
                buf = rows_v.at[b]
                pltpu.make_async_copy(z_hbm.at[src_v.at[j]], buf, gsems[b]).wait()
                jn = j + NBUF - 1
                bn = (b + NBUF - 1) % NBUF

                @pl.when(jn < nchunk)
                def _issue():
                    pltpu.async_copy(z_hbm.at[src_v.at[jn]], rows_v.at[bn],
                                     gsems[bn])

                @pl.loop(0, CHUNK // 16)
                def _scale(eg):
                    wv = w_v[j, pl.ds(eg * 16, 16)]
                    for l in range(16):
                        ws = wv[l]
                        e = eg * 16 + l
                        for gg in range(NOUT // 16):
                            sl = pl.ds(gg * 16, 16)
                            buf[e, sl] = buf[e, sl] * ws

                pltpu.sync_copy(buf, acc_sh.at[dst_v.at[j]], add=True)

        plsc.subcore_barrier()
        for r in range(ROWS_PER_TILE // WB):
            sl = pl.ds(row0 + r * WB, WB)
            pltpu.sync_copy(acc_sh.at[sl], out_hbm.at[cid, sl])

    return k(z, src_r, dst_r, w_r)


def _combine(acc, b2):
    def body(a_ref, b2_ref, o_ref):
        o_ref[...] = a_ref[0, :N_NODES] + a_ref[1, :N_NODES] + b2_ref[...]

    return pl.pallas_call(
        body,
        out_shape=jax.ShapeDtypeStruct((N_NODES, NOUT), jnp.float32),
    )(acc, b2.reshape(1, -1))


def kernel(x, edge_index, edge_weight, W1, b1, W2, b2):
    e = edge_weight.shape[0]
    per_tile = -(-e // (NW * CHUNK * NBUF)) * (CHUNK * NBUF)
    pad = per_tile * NW - e
    src = edge_index[1].astype(jnp.int32)
    dst = edge_index[0].astype(jnp.int32)
    w = edge_weight.astype(jnp.float32)
    if pad:
        src = jnp.concatenate([src, jnp.zeros((pad,), jnp.int32)])
        dst = jnp.concatenate([dst, jnp.zeros((pad,), jnp.int32)])
        w = jnp.concatenate([w, jnp.zeros((pad,), jnp.float32)])
    nchunk = per_tile // CHUNK
    src_r = src.reshape(NW, nchunk, CHUNK)
    dst_r = dst.reshape(NW, nchunk, CHUNK)
    w_r = w.reshape(NW, nchunk, CHUNK)

    z = _dense_z(x, W1, b1, W2)
    acc = _sc_segsum(z, src_r, dst_r, w_r, nchunk)
    return _combine(acc, b2)


# speedup vs baseline: 7.8915x; 1.0023x over previous
import functools

import jax
import jax.numpy as jnp
from jax import lax
from jax.experimental import pallas as pl
from jax.experimental.pallas import tpu as pltpu
from jax.experimental.pallas import tpu_sc as plsc

N_NODES = 10000
NOUT = 64

NUM_CORES = 2
NUM_SUBCORES = 16
NW = NUM_CORES * NUM_SUBCORES
CHUNK = 128
N_PAD = 10240
ROWS_PER_TILE = N_PAD // NUM_SUBCORES
WB = 128


def _dense_z(x, W1, b1, W2):
    def body(x_ref, w1_ref, b1_ref, w2_ref, z_ref):
        m = jnp.dot(w1_ref[...], w2_ref[...], preferred_element_type=jnp.float32)
        v = jnp.dot(b1_ref[...], w2_ref[...], preferred_element_type=jnp.float32)
        z_ref[...] = jnp.dot(x_ref[...], m, preferred_element_type=jnp.float32) + v

    return pl.pallas_call(
        body,
        out_shape=jax.ShapeDtypeStruct((x.shape[0], W2.shape[1]), jnp.float32),
    )(x, W1, b1.reshape(1, -1), W2)


NBUF = 4


def _sc_segsum(z, src_r, dst_r, w_r, nchunk):
    mesh = plsc.VectorSubcoreMesh(core_axis_name="c", subcore_axis_name="s")
    assert nchunk % NBUF == 0

    @functools.partial(
        pl.kernel,
        out_type=jax.ShapeDtypeStruct((NUM_CORES, N_PAD, NOUT), jnp.float32),
        mesh=mesh,
        scratch_types=[
            pltpu.VMEM((nchunk, CHUNK), jnp.int32),
            pltpu.VMEM((nchunk, CHUNK), jnp.int32),
            pltpu.VMEM((nchunk, CHUNK), jnp.float32),
            pltpu.VMEM((NBUF, CHUNK, NOUT), jnp.float32),
            pltpu.VMEM((WB, NOUT), jnp.float32),
            pltpu.VMEM_SHARED((N_PAD, NOUT), jnp.float32),
            [pltpu.SemaphoreType.DMA] * NBUF,
            [pltpu.SemaphoreType.DMA] * NBUF,
        ],
        compiler_params=pltpu.CompilerParams(use_tc_tiling_on_sc=False),
    )
    def k(z_hbm, src_hbm, dst_hbm, w_hbm, out_hbm,
          src_v, dst_v, w_v, rows_v, zero_v, acc_sh, gsems, ssems):
        cid = lax.axis_index("c")
        sid = lax.axis_index("s")
        tid = cid * NUM_SUBCORES + sid

        @pl.loop(0, WB)
        def _zero_rows(i):
            for g in range(NOUT // 16):
                zero_v[i, pl.ds(g * 16, 16)] = jnp.zeros((16,), jnp.float32)

        row0 = sid * ROWS_PER_TILE
        for r in range(ROWS_PER_TILE // WB):
            pltpu.sync_copy(zero_v, acc_sh.at[pl.ds(row0 + r * WB, WB)])
        plsc.subcore_barrier()

        pltpu.sync_copy(src_hbm.at[tid], src_v)
        pltpu.sync_copy(dst_hbm.at[tid], dst_v)
        pltpu.sync_copy(w_hbm.at[tid], w_v)

        for b in range(NBUF - 1):
            pltpu.async_copy(z_hbm.at[src_v.at[b]], rows_v.at[b], gsems[b])

        ngroups = nchunk // NBUF

        @pl.loop(0, ngroups)
        def _groups(g):
            for b in range(NBUF):
                j = g * NBUF + b
                buf = rows_v.at[b]
                pltpu.make_async_copy(z_hbm.at[src_v.at[j]], buf, gsems[b]).wait()

                @pl.loop(0, CHUNK // 16)
                def _scale(eg):
                    wv = w_v[j, pl.ds(eg * 16, 16)]
                    for l in range(16):
                        ws = wv[l]
                        e = eg * 16 + l
                        for gg in range(NOUT // 16):
                            sl = pl.ds(gg * 16, 16)
                            buf[e, sl] = buf[e, sl] * ws

                pltpu.async_copy(buf, acc_sh.at[dst_v.at[j]], ssems[b],
                                 add=True)

                jn = j + NBUF - 1
                bn = (b + NBUF - 1) % NBUF

                def _recycle():
                    pltpu.make_async_copy(
                        rows_v.at[bn], acc_sh.at[dst_v.at[jn - NBUF]],
                        ssems[bn]).wait()
                    pltpu.async_copy(z_hbm.at[src_v.at[jn]], rows_v.at[bn],
                                     gsems[bn])

                if b == 0:
                    @pl.when(g > 0)
                    def _():
                        _recycle()

                    @pl.when(g == 0)
                    def _():
                        pltpu.async_copy(z_hbm.at[src_v.at[jn]],
                                         rows_v.at[bn], gsems[bn])
                else:
                    @pl.when(g < ngroups - 1)
                    def _():
                        _recycle()

        for b in range(NBUF):
            j = nchunk - NBUF + b
            pltpu.make_async_copy(rows_v.at[b], acc_sh.at[dst_v.at[j]],
                                  ssems[b]).wait()
        plsc.subcore_barrier()
        for r in range(ROWS_PER_TILE // WB):
            sl = pl.ds(row0 + r * WB, WB)
            pltpu.sync_copy(acc_sh.at[sl], out_hbm.at[cid, sl])

    return k(z, src_r, dst_r, w_r)


def _combine(acc, b2):
    def body(a_ref, b2_ref, o_ref):
        o_ref[...] = a_ref[0, :N_NODES] + a_ref[1, :N_NODES] + b2_ref[...]

    return pl.pallas_call(
        body,
        out_shape=jax.ShapeDtypeStruct((N_NODES, NOUT), jnp.float32),
    )(acc, b2.reshape(1, -1))


def kernel(x, edge_index, edge_weight, W1, b1, W2, b2):
    e = edge_weight.shape[0]
    per_tile = -(-e // (NW * CHUNK * NBUF)) * (CHUNK * NBUF)
    pad = per_tile * NW - e
    src = edge_index[1].astype(jnp.int32)
    dst = edge_index[0].astype(jnp.int32)
    w = edge_weight.astype(jnp.float32)
    if pad:
        src = jnp.concatenate([src, jnp.zeros((pad,), jnp.int32)])
        dst = jnp.concatenate([dst, jnp.zeros((pad,), jnp.int32)])
        w = jnp.concatenate([w, jnp.zeros((pad,), jnp.float32)])
    nchunk = per_tile // CHUNK
    src_r = src.reshape(NW, nchunk, CHUNK)
    dst_r = dst.reshape(NW, nchunk, CHUNK)
    w_r = w.reshape(NW, nchunk, CHUNK)

    z = _dense_z(x, W1, b1, W2)
    acc = _sc_segsum(z, src_r, dst_r, w_r, nchunk)
    return _combine(acc, b2)


# speedup vs baseline: 17.1022x; 2.1672x over previous
import functools

import jax
import jax.numpy as jnp
from jax import lax
from jax.experimental import pallas as pl
from jax.experimental.pallas import tpu as pltpu
from jax.experimental.pallas import tpu_sc as plsc

N_NODES = 10000
NOUT = 64

NUM_CORES = 2
NUM_SUBCORES = 16
NH = NOUT // NUM_CORES
CHUNK = 128
N_PAD = 10240
ROWS_PER_TILE = N_PAD // NUM_SUBCORES
WB = 128
NBUF = 4


def _dense_z(x, W1, b1, W2):
    def body(x_ref, w1_ref, b1_ref, w2_ref, z_ref):
        m = jnp.dot(w1_ref[...], w2_ref[...], preferred_element_type=jnp.float32)
        v = jnp.dot(b1_ref[...], w2_ref[...], preferred_element_type=jnp.float32)
        res = jnp.dot(x_ref[...], m, preferred_element_type=jnp.float32) + v
        z_ref[0, :N_NODES] = res[:, :NH]
        z_ref[1, :N_NODES] = res[:, NH:]

    return pl.pallas_call(
        body,
        out_shape=jax.ShapeDtypeStruct((NUM_CORES, N_PAD, NH), jnp.float32),
    )(x, W1, b1.reshape(1, -1), W2)


def _sc_segsum(z, src_r, dst_r, w_r, nchunk):
    mesh = plsc.VectorSubcoreMesh(core_axis_name="c", subcore_axis_name="s")
    assert nchunk % NBUF == 0

    @functools.partial(
        pl.kernel,
        out_type=jax.ShapeDtypeStruct((NUM_CORES, N_PAD, NH), jnp.float32),
        mesh=mesh,
        scratch_types=[
            pltpu.VMEM((nchunk, CHUNK), jnp.int32),
            pltpu.VMEM((nchunk, CHUNK), jnp.int32),
            pltpu.VMEM((nchunk, CHUNK), jnp.float32),
            pltpu.VMEM((NBUF, CHUNK, NH), jnp.float32),
            pltpu.VMEM((WB, NH), jnp.float32),
            pltpu.VMEM_SHARED((N_PAD, NH), jnp.float32),
            pltpu.VMEM_SHARED((N_PAD, NH), jnp.float32),
            [pltpu.SemaphoreType.DMA] * NBUF,
            [pltpu.SemaphoreType.DMA] * NBUF,
        ],
        compiler_params=pltpu.CompilerParams(use_tc_tiling_on_sc=False),
    )
    def k(z_hbm, src_hbm, dst_hbm, w_hbm, out_hbm,
          src_v, dst_v, w_v, rows_v, zero_v, acc_sh, z_sh, gsems, ssems):
        cid = lax.axis_index("c")
        sid = lax.axis_index("s")

        @pl.loop(0, WB)
        def _zero_rows(i):
            for g in range(NH // 16):
                zero_v[i, pl.ds(g * 16, 16)] = jnp.zeros((16,), jnp.float32)

        row0 = sid * ROWS_PER_TILE
        for r in range(ROWS_PER_TILE // WB):
            pltpu.sync_copy(zero_v, acc_sh.at[pl.ds(row0 + r * WB, WB)])
        sl0 = pl.ds(row0, ROWS_PER_TILE)
        pltpu.sync_copy(z_hbm.at[cid, sl0], z_sh.at[sl0])
        plsc.subcore_barrier()

        pltpu.sync_copy(src_hbm.at[sid], src_v)
        pltpu.sync_copy(dst_hbm.at[sid], dst_v)
        pltpu.sync_copy(w_hbm.at[sid], w_v)

        for b in range(NBUF - 1):
            pltpu.async_copy(z_sh.at[src_v.at[b]], rows_v.at[b], gsems[b])

        ngroups = nchunk // NBUF

        @pl.loop(0, ngroups)
        def _groups(g):
            for b in range(NBUF):
                j = g * NBUF + b
                buf = rows_v.at[b]
                pltpu.make_async_copy(z_sh.at[src_v.at[j]], buf, gsems[b]).wait()

                @pl.loop(0, CHUNK // 16)
                def _scale(eg):
                    wv = w_v[j, pl.ds(eg * 16, 16)]
                    for l in range(16):
                        ws = wv[l]
                        e = eg * 16 + l
                        for gg in range(NH // 16):
                            sl = pl.ds(gg * 16, 16)
                            buf[e, sl] = buf[e, sl] * ws

                pltpu.async_copy(buf, acc_sh.at[dst_v.at[j]], ssems[b],
                                 add=True)

                jn = j + NBUF - 1
                bn = (b + NBUF - 1) % NBUF

                def _recycle():
                    pltpu.make_async_copy(
                        rows_v.at[bn], acc_sh.at[dst_v.at[jn - NBUF]],
                        ssems[bn]).wait()
                    pltpu.async_copy(z_sh.at[src_v.at[jn]], rows_v.at[bn],
                                     gsems[bn])

                if b == 0:
                    @pl.when(g > 0)
                    def _():
                        _recycle()

                    @pl.when(g == 0)
                    def _():
                        pltpu.async_copy(z_sh.at[src_v.at[jn]],
                                        rows_v.at[bn], gsems[bn])
                else:
                    @pl.when(g < ngroups - 1)
                    def _():
                        _recycle()

        for b in range(NBUF):
            j = nchunk - NBUF + b
            pltpu.make_async_copy(rows_v.at[b], acc_sh.at[dst_v.at[j]],
                                  ssems[b]).wait()
        plsc.subcore_barrier()
        for r in range(ROWS_PER_TILE // WB):
            sl = pl.ds(row0 + r * WB, WB)
            pltpu.sync_copy(acc_sh.at[sl], out_hbm.at[cid, sl])

    return k(z, src_r, dst_r, w_r)


def _combine(acc, b2):
    def body(a_ref, b2_ref, o_ref):
        o_ref[...] = jnp.concatenate(
            [a_ref[0, :N_NODES], a_ref[1, :N_NODES]], axis=1) + b2_ref[...]

    return pl.pallas_call(
        body,
        out_shape=jax.ShapeDtypeStruct((N_NODES, NOUT), jnp.float32),
    )(acc, b2.reshape(1, -1))


def kernel(x, edge_index, edge_weight, W1, b1, W2, b2):
    e = edge_weight.shape[0]
    per_tile = -(-e // (NUM_SUBCORES * CHUNK * NBUF)) * (CHUNK * NBUF)
    pad = per_tile * NUM_SUBCORES - e
    src = edge_index[1].astype(jnp.int32)
    dst = edge_index[0].astype(jnp.int32)
    w = edge_weight.astype(jnp.float32)
    if pad:
        src = jnp.concatenate([src, jnp.zeros((pad,), jnp.int32)])
        dst = jnp.concatenate([dst, jnp.zeros((pad,), jnp.int32)])
        w = jnp.concatenate([w, jnp.zeros((pad,), jnp.float32)])
    nchunk = per_tile // CHUNK
    src_r = src.reshape(NUM_SUBCORES, nchunk, CHUNK)
    dst_r = dst.reshape(NUM_SUBCORES, nchunk, CHUNK)
    w_r = w.reshape(NUM_SUBCORES, nchunk, CHUNK)

    z = _dense_z(x, W1, b1, W2)
    acc = _sc_segsum(z, src_r, dst_r, w_r, nchunk)
    return _combine(acc, b2)
